# baseline (speedup 1.0000x reference)
import functools

import jax
import jax.numpy as jnp
from jax import lax
from jax.experimental import pallas as pl
from jax.experimental.pallas import tpu as pltpu
from jax.experimental.pallas import tpu_sc as plsc

BATCH = 16384
EMBED = 64
CHUNK = 128
LANES = 16


def _sc_body(uidx_hbm, iidx_hbm, emb_hbm, ubias_hbm, ibias_hbm, ob_hbm,
             out_hbm,
             uidx_v, iidx_v, u_rows, v_rows, ub_v, ib_v, ob_v, out_v, sem):
  nc = 2
  wid = lax.axis_index("s") * nc + lax.axis_index("c")
  n_per = BATCH // 32
  n_chunks = n_per // CHUNK
  base = wid * n_per

  row0 = wid * n_chunks
  pltpu.sync_copy(uidx_hbm.at[pl.ds(row0, n_chunks)], uidx_v)
  pltpu.sync_copy(iidx_hbm.at[pl.ds(row0, n_chunks)], iidx_v)
  pltpu.sync_copy(ob_hbm, ob_v)

  copies = []
  for c in range(n_chunks):
    sl = pl.ds(c * CHUNK, CHUNK)
    copies.append(pltpu.make_async_copy(emb_hbm.at[uidx_v.at[c]],
                                        u_rows.at[sl], sem))
    copies.append(pltpu.make_async_copy(emb_hbm.at[iidx_v.at[c]],
                                        v_rows.at[sl], sem))
    copies.append(pltpu.make_async_copy(ubias_hbm.at[uidx_v.at[c]],
                                        ub_v.at[sl], sem))
    copies.append(pltpu.make_async_copy(ibias_hbm.at[iidx_v.at[c]],
                                        ib_v.at[sl], sem))
  for cp in copies:
    cp.start()
  for cp in copies:
    cp.wait()

  ob = ob_v[...]
  lane = lax.iota(jnp.int32, LANES)
  zeros16 = jnp.zeros((LANES,), jnp.int32)

  def group_body(g, _):
    rows = g * LANES + lane
    acc = jnp.zeros((LANES,), jnp.float32)

    def f_body(f, acc):
      cols = zeros16 + f
      u = plsc.load_gather(u_rows, [rows, cols])
      v = plsc.load_gather(v_rows, [rows, cols])
      return acc + u * v

    acc = lax.fori_loop(0, EMBED, f_body, acc, unroll=8)
    sl = pl.ds(g * LANES, LANES)
    out_v[sl] = acc + ub_v[sl] + ib_v[sl] + ob
    return 0

  lax.fori_loop(0, n_per // LANES, group_body, 0)

  pltpu.sync_copy(out_v, out_hbm.at[pl.ds(base, n_per)])


@jax.jit
def _run(uidx2d, iidx2d, emb, ubias1d, ibias1d, ob16):
  n_per = BATCH // 32
  mesh = plsc.VectorSubcoreMesh(core_axis_name="c", subcore_axis_name="s")
  f = pl.kernel(
      _sc_body,
      out_type=jax.ShapeDtypeStruct((BATCH,), jnp.float32),
      mesh=mesh,
      scratch_types=[
          pltpu.VMEM((n_per // CHUNK, CHUNK), jnp.int32),
          pltpu.VMEM((n_per // CHUNK, CHUNK), jnp.int32),
          pltpu.VMEM((n_per, EMBED), jnp.float32),
          pltpu.VMEM((n_per, EMBED), jnp.float32),
          pltpu.VMEM((n_per,), jnp.float32),
          pltpu.VMEM((n_per,), jnp.float32),
          pltpu.VMEM((LANES,), jnp.float32),
          pltpu.VMEM((n_per,), jnp.float32),
          pltpu.SemaphoreType.DMA,
      ],
      compiler_params=pltpu.CompilerParams(
          needs_layout_passes=False, use_tc_tiling_on_sc=False),
  )
  return f(uidx2d, iidx2d, emb, ubias1d, ibias1d, ob16)


def kernel(user_idx, item_idx, userEmbd, userBias, itemBias, overall_bias):
  uidx2d = user_idx.astype(jnp.int32).reshape(BATCH // CHUNK, CHUNK)
  iidx2d = item_idx.astype(jnp.int32).reshape(BATCH // CHUNK, CHUNK)
  ubias1d = userBias.reshape(-1)
  ibias1d = itemBias.reshape(-1)
  ob16 = jnp.broadcast_to(overall_bias.astype(jnp.float32), (LANES,))
  return _run(uidx2d, iidx2d, userEmbd, ubias1d, ibias1d, ob16)


# speedup vs baseline: 1.0012x; 1.0012x over previous
import jax
import jax.numpy as jnp
from jax import lax
from jax.experimental import pallas as pl
from jax.experimental.pallas import tpu as pltpu
from jax.experimental.pallas import tpu_sc as plsc

BATCH = 16384
EMBED = 64
CHUNK = 128
LANES = 16
N_PER = BATCH // 32
N_CHUNKS = N_PER // CHUNK


def _sc_body(uidx_hbm, iidx_hbm, emb2_hbm, ubias_hbm, ibias_hbm, ob_hbm,
             out_hbm,
             uidx_v, iidx_v, upair_v, ipair_v, u_buf, v_buf,
             ub_v, ib_v, ob_v, out_v, bias_sem, usem, vsem):
  nc = 2
  wid = lax.axis_index("s") * nc + lax.axis_index("c")
  base = wid * N_PER

  pltpu.sync_copy(uidx_hbm.at[pl.ds(base, N_PER)], uidx_v)
  pltpu.sync_copy(iidx_hbm.at[pl.ds(base, N_PER)], iidx_v)
  pltpu.sync_copy(ob_hbm, ob_v)

  def pair_body(j, _):
    sl = pl.ds(j * LANES, LANES)
    upair_v[sl] = lax.shift_right_logical(uidx_v[sl], 1)
    ipair_v[sl] = lax.shift_right_logical(iidx_v[sl], 1)
    return 0

  lax.fori_loop(0, N_PER // LANES, pair_body, 0)

  for c in range(N_CHUNKS):
    sl = pl.ds(c * CHUNK, CHUNK)
    pltpu.make_async_copy(ubias_hbm.at[uidx_v.at[sl]], ub_v.at[sl],
                          bias_sem).start()
    pltpu.make_async_copy(ibias_hbm.at[iidx_v.at[sl]], ib_v.at[sl],
                          bias_sem).start()

  def fire(c, buf):
    sl = pl.ds(c * CHUNK, CHUNK)
    pltpu.make_async_copy(emb2_hbm.at[upair_v.at[sl]], u_buf.at[buf],
                          usem).start()
    pltpu.make_async_copy(emb2_hbm.at[ipair_v.at[sl]], v_buf.at[buf],
                          vsem).start()

  def wait(buf):
    pltpu.make_async_copy(emb2_hbm.at[upair_v.at[pl.ds(0, CHUNK)]],
                          u_buf.at[buf], usem).wait()
    pltpu.make_async_copy(emb2_hbm.at[ipair_v.at[pl.ds(0, CHUNK)]],
                          v_buf.at[buf], vsem).wait()

  lane = lax.iota(jnp.int32, LANES)

  def compute(c, buf):
    def group_body(g, _):
      sl = pl.ds(c * CHUNK + g * LANES, LANES)
      rows = g * LANES + lane
      ucols0 = (uidx_v[sl] & 1) * EMBED
      icols0 = (iidx_v[sl] & 1) * EMBED
      acc = jnp.zeros((LANES,), jnp.float32)

      def f_body(f, acc):
        u = plsc.load_gather(u_buf.at[buf], [rows, ucols0 + f])
        v = plsc.load_gather(v_buf.at[buf], [rows, icols0 + f])
        return acc + u * v

      acc = lax.fori_loop(0, EMBED, f_body, acc, unroll=8)
      out_v[sl] = acc
      return 0

    lax.fori_loop(0, CHUNK // LANES, group_body, 0)

  fire(0, 0)
  for c in range(N_CHUNKS):
    if c + 1 < N_CHUNKS:
      fire(c + 1, (c + 1) % 2)
    wait(c % 2)
    compute(c, c % 2)

  for c in range(N_CHUNKS):
    sl = pl.ds(c * CHUNK, CHUNK)
    pltpu.make_async_copy(ubias_hbm.at[uidx_v.at[sl]], ub_v.at[sl],
                          bias_sem).wait()
    pltpu.make_async_copy(ibias_hbm.at[iidx_v.at[sl]], ib_v.at[sl],
                          bias_sem).wait()
  ob = ob_v[...]

  def bias_body(j, _):
    sl = pl.ds(j * LANES, LANES)
    out_v[sl] = out_v[sl] + ub_v[sl] + ib_v[sl] + ob
    return 0

  lax.fori_loop(0, N_PER // LANES, bias_body, 0)

  pltpu.sync_copy(out_v, out_hbm.at[pl.ds(base, N_PER)])


@jax.jit
def _run(uidx, iidx, emb2, ubias1d, ibias1d, ob16):
  mesh = plsc.VectorSubcoreMesh(core_axis_name="c", subcore_axis_name="s")
  f = pl.kernel(
      _sc_body,
      out_type=jax.ShapeDtypeStruct((BATCH,), jnp.float32),
      mesh=mesh,
      scratch_types=[
          pltpu.VMEM((N_PER,), jnp.int32),
          pltpu.VMEM((N_PER,), jnp.int32),
          pltpu.VMEM((N_PER,), jnp.int32),
          pltpu.VMEM((N_PER,), jnp.int32),
          pltpu.VMEM((2, CHUNK, 2 * EMBED), jnp.float32),
          pltpu.VMEM((2, CHUNK, 2 * EMBED), jnp.float32),
          pltpu.VMEM((N_PER,), jnp.float32),
          pltpu.VMEM((N_PER,), jnp.float32),
          pltpu.VMEM((LANES,), jnp.float32),
          pltpu.VMEM((N_PER,), jnp.float32),
          pltpu.SemaphoreType.DMA,
          pltpu.SemaphoreType.DMA,
          pltpu.SemaphoreType.DMA,
      ],
      compiler_params=pltpu.CompilerParams(
          needs_layout_passes=False, use_tc_tiling_on_sc=True),
  )
  return f(uidx, iidx, emb2, ubias1d, ibias1d, ob16)


def kernel(user_idx, item_idx, userEmbd, userBias, itemBias, overall_bias):
  uidx = user_idx.astype(jnp.int32)
  iidx = item_idx.astype(jnp.int32)
  emb2 = userEmbd.reshape(1000000 // 2, 2 * EMBED)
  ubias1d = userBias.reshape(-1)
  ibias1d = itemBias.reshape(-1)
  ob16 = jnp.broadcast_to(overall_bias.astype(jnp.float32), (LANES,))
  return _run(uidx, iidx, emb2, ubias1d, ibias1d, ob16)


# speedup vs baseline: 1.3076x; 1.3060x over previous
import jax
import jax.numpy as jnp
from jax import lax
from jax.experimental import pallas as pl
from jax.experimental.pallas import tpu as pltpu
from jax.experimental.pallas import tpu_sc as plsc

BATCH = 16384
EMBED = 64
LANES = 16
N_PER = BATCH // 32
WAVE = 16
N_WAVES = N_PER // WAVE
BCHUNK = 128


def _sc_body(uidx_hbm, iidx_hbm, emb_hbm, ubias_hbm, ibias_hbm, ob_hbm,
             out_hbm,
             uidx_v, iidx_v, u_blk, v_blk,
             ub_v, ib_v, ob_v, out_v, bias_sem, usem, vsem):
  nc = 2
  wid = lax.axis_index("s") * nc + lax.axis_index("c")
  base = wid * N_PER

  pltpu.sync_copy(uidx_hbm.at[pl.ds(base, N_PER)], uidx_v)
  pltpu.sync_copy(iidx_hbm.at[pl.ds(base, N_PER)], iidx_v)
  pltpu.sync_copy(ob_hbm, ob_v)

  for c in range(N_PER // BCHUNK):
    sl = pl.ds(c * BCHUNK, BCHUNK)
    pltpu.make_async_copy(ubias_hbm.at[uidx_v.at[sl]], ub_v.at[sl],
                          bias_sem).start()
    pltpu.make_async_copy(ibias_hbm.at[iidx_v.at[sl]], ib_v.at[sl],
                          bias_sem).start()

  def fire(w, buf):
    sl = pl.ds(w * WAVE, WAVE)
    uq = lax.shift_right_logical(uidx_v[sl], 3) * 8
    iq = lax.shift_right_logical(iidx_v[sl], 3) * 8
    for j in range(WAVE):
      rqu = pl.multiple_of(uq[j], 8)
      rqi = pl.multiple_of(iq[j], 8)
      pltpu.make_async_copy(emb_hbm.at[pl.ds(rqu, 8)], u_blk.at[buf, j],
                            usem).start()
      pltpu.make_async_copy(emb_hbm.at[pl.ds(rqi, 8)], v_blk.at[buf, j],
                            vsem).start()

  def wait(buf):
    def lp(j, _):
      pltpu.make_async_copy(emb_hbm.at[pl.ds(0, 8)], u_blk.at[buf, j],
                            usem).wait()
      pltpu.make_async_copy(emb_hbm.at[pl.ds(0, 8)], v_blk.at[buf, j],
                            vsem).wait()
      return 0

    lax.fori_loop(0, WAVE, lp, 0)

  lane = lax.iota(jnp.int32, LANES)

  def compute(w, buf):
    def group_body(g, _):
      sl = pl.ds(w * WAVE + g * LANES, LANES)
      blk = g * LANES + lane
      usub = uidx_v[sl] & 7
      isub = iidx_v[sl] & 7
      acc = jnp.zeros((LANES,), jnp.float32)

      def f_body(f, acc):
        cols = jnp.zeros((LANES,), jnp.int32) + f
        u = plsc.load_gather(u_blk.at[buf], [blk, usub, cols])
        v = plsc.load_gather(v_blk.at[buf], [blk, isub, cols])
        return acc + u * v

      acc = lax.fori_loop(0, EMBED, f_body, acc, unroll=8)
      out_v[sl] = acc
      return 0

    lax.fori_loop(0, WAVE // LANES, group_body, 0)

  fire(0, 0)

  def wave_body(w, _):
    @pl.when(w + 1 < N_WAVES)
    def _():
      fire(w + 1, (w + 1) % 2)
    wait(w % 2)
    compute(w, w % 2)
    return 0

  lax.fori_loop(0, N_WAVES, wave_body, 0)

  for c in range(N_PER // BCHUNK):
    sl = pl.ds(c * BCHUNK, BCHUNK)
    pltpu.make_async_copy(ubias_hbm.at[uidx_v.at[sl]], ub_v.at[sl],
                          bias_sem).wait()
    pltpu.make_async_copy(ibias_hbm.at[iidx_v.at[sl]], ib_v.at[sl],
                          bias_sem).wait()
  ob = ob_v[...]

  def bias_body(j, _):
    sl = pl.ds(j * LANES, LANES)
    out_v[sl] = out_v[sl] + ub_v[sl] + ib_v[sl] + ob
    return 0

  lax.fori_loop(0, N_PER // LANES, bias_body, 0)

  pltpu.sync_copy(out_v, out_hbm.at[pl.ds(base, N_PER)])


@jax.jit
def _run(uidx, iidx, emb, ubias1d, ibias1d, ob16):
  mesh = plsc.VectorSubcoreMesh(core_axis_name="c", subcore_axis_name="s")
  f = pl.kernel(
      _sc_body,
      out_type=jax.ShapeDtypeStruct((BATCH,), jnp.float32),
      mesh=mesh,
      scratch_types=[
          pltpu.VMEM((N_PER,), jnp.int32),
          pltpu.VMEM((N_PER,), jnp.int32),
          pltpu.VMEM((2, WAVE, 8, EMBED), jnp.float32),
          pltpu.VMEM((2, WAVE, 8, EMBED), jnp.float32),
          pltpu.VMEM((N_PER,), jnp.float32),
          pltpu.VMEM((N_PER,), jnp.float32),
          pltpu.VMEM((LANES,), jnp.float32),
          pltpu.VMEM((N_PER,), jnp.float32),
          pltpu.SemaphoreType.DMA,
          pltpu.SemaphoreType.DMA,
          pltpu.SemaphoreType.DMA,
      ],
      compiler_params=pltpu.CompilerParams(
          needs_layout_passes=False, use_tc_tiling_on_sc=True),
  )
  return f(uidx, iidx, emb, ubias1d, ibias1d, ob16)


def kernel(user_idx, item_idx, userEmbd, userBias, itemBias, overall_bias):
  uidx = user_idx.astype(jnp.int32)
  iidx = item_idx.astype(jnp.int32)
  ubias1d = userBias.reshape(-1)
  ibias1d = itemBias.reshape(-1)
  ob16 = jnp.broadcast_to(overall_bias.astype(jnp.float32), (LANES,))
  return _run(uidx, iidx, userEmbd, ubias1d, ibias1d, ob16)


# speedup vs baseline: 1.5842x; 1.2116x over previous
import jax
import jax.numpy as jnp
from jax import lax
from jax.experimental import pallas as pl
from jax.experimental.pallas import tpu as pltpu
from jax.experimental.pallas import tpu_sc as plsc

BATCH = 16384
EMBED = 64
LANES = 16
N_PER = BATCH // 32
WAVE = 16
N_WAVES = N_PER // WAVE
BCHUNK = 128


def _sc_body(uidx_hbm, iidx_hbm, emb_hbm, ubias_hbm, ibias_hbm, ob_hbm,
             out_hbm,
             uidx_v, iidx_v, u_blk, v_blk,
             ub_v, ib_v, ob_v, out_v, bias_sem, usem, vsem):
  nc = 2
  wid = lax.axis_index("s") * nc + lax.axis_index("c")
  base = wid * N_PER

  pltpu.sync_copy(uidx_hbm.at[pl.ds(base, N_PER)], uidx_v)
  pltpu.sync_copy(iidx_hbm.at[pl.ds(base, N_PER)], iidx_v)
  pltpu.sync_copy(ob_hbm, ob_v)

  for c in range(N_PER // BCHUNK):
    sl = pl.ds(c * BCHUNK, BCHUNK)
    pltpu.make_async_copy(ubias_hbm.at[0].at[uidx_v.at[sl]], ub_v.at[sl],
                          bias_sem).start()
    pltpu.make_async_copy(ibias_hbm.at[0].at[iidx_v.at[sl]], ib_v.at[sl],
                          bias_sem).start()

  def fire(w, buf):
    sl = pl.ds(w * WAVE, WAVE)
    uq = lax.shift_right_logical(uidx_v[sl], 3) * 8
    iq = lax.shift_right_logical(iidx_v[sl], 3) * 8
    for j in range(WAVE):
      rqu = pl.multiple_of(uq[j], 8)
      rqi = pl.multiple_of(iq[j], 8)
      pltpu.make_async_copy(emb_hbm.at[pl.ds(rqu, 8)], u_blk.at[buf, j],
                            usem).start()
      pltpu.make_async_copy(emb_hbm.at[pl.ds(rqi, 8)], v_blk.at[buf, j],
                            vsem).start()

  def wait(buf):
    def lp(j, _):
      pltpu.make_async_copy(emb_hbm.at[pl.ds(0, 8)], u_blk.at[buf, j],
                            usem).wait()
      pltpu.make_async_copy(emb_hbm.at[pl.ds(0, 8)], v_blk.at[buf, j],
                            vsem).wait()
      return 0

    lax.fori_loop(0, WAVE, lp, 0)

  lane = lax.iota(jnp.int32, LANES)

  def compute(w, buf):
    def group_body(g, _):
      sl = pl.ds(w * WAVE + g * LANES, LANES)
      blk = g * LANES + lane
      usub = uidx_v[sl] & 7
      isub = iidx_v[sl] & 7
      acc = jnp.zeros((LANES,), jnp.float32)

      def f_body(f, acc):
        cols = jnp.zeros((LANES,), jnp.int32) + f
        u = plsc.load_gather(u_blk.at[buf], [blk, usub, cols])
        v = plsc.load_gather(v_blk.at[buf], [blk, isub, cols])
        return acc + u * v

      acc = lax.fori_loop(0, EMBED, f_body, acc, unroll=8)
      out_v[sl] = acc
      return 0

    lax.fori_loop(0, WAVE // LANES, group_body, 0)

  fire(0, 0)

  def wave_body(w, _):
    @pl.when(w + 1 < N_WAVES)
    def _():
      fire(w + 1, (w + 1) % 2)
    wait(w % 2)
    compute(w, w % 2)
    return 0

  lax.fori_loop(0, N_WAVES, wave_body, 0)

  for c in range(N_PER // BCHUNK):
    sl = pl.ds(c * BCHUNK, BCHUNK)
    pltpu.make_async_copy(ubias_hbm.at[0].at[uidx_v.at[sl]], ub_v.at[sl],
                          bias_sem).wait()
    pltpu.make_async_copy(ibias_hbm.at[0].at[iidx_v.at[sl]], ib_v.at[sl],
                          bias_sem).wait()
  ob = ob_v[...]

  def bias_body(j, _):
    sl = pl.ds(j * LANES, LANES)
    out_v[sl] = out_v[sl] + ub_v[sl] + ib_v[sl] + ob
    return 0

  lax.fori_loop(0, N_PER // LANES, bias_body, 0)

  pltpu.sync_copy(out_v, out_hbm.at[pl.ds(base, N_PER)])


@jax.jit
def _run(uidx, iidx, emb, ubias2, ibias2, ob16):
  mesh = plsc.VectorSubcoreMesh(core_axis_name="c", subcore_axis_name="s")
  f = pl.kernel(
      _sc_body,
      out_type=jax.ShapeDtypeStruct((BATCH,), jnp.float32),
      mesh=mesh,
      scratch_types=[
          pltpu.VMEM((N_PER,), jnp.int32),
          pltpu.VMEM((N_PER,), jnp.int32),
          pltpu.VMEM((2, WAVE, 8, EMBED), jnp.float32),
          pltpu.VMEM((2, WAVE, 8, EMBED), jnp.float32),
          pltpu.VMEM((N_PER,), jnp.float32),
          pltpu.VMEM((N_PER,), jnp.float32),
          pltpu.VMEM((LANES,), jnp.float32),
          pltpu.VMEM((N_PER,), jnp.float32),
          pltpu.SemaphoreType.DMA,
          pltpu.SemaphoreType.DMA,
          pltpu.SemaphoreType.DMA,
      ],
      compiler_params=pltpu.CompilerParams(
          needs_layout_passes=False, use_tc_tiling_on_sc=True),
  )
  return f(uidx, iidx, emb, ubias2, ibias2, ob16)


def kernel(user_idx, item_idx, userEmbd, userBias, itemBias, overall_bias):
  uidx = user_idx.astype(jnp.int32)
  iidx = item_idx.astype(jnp.int32)
  ubias2 = userBias.T
  ibias2 = itemBias.T
  ob16 = jnp.broadcast_to(overall_bias.astype(jnp.float32), (LANES,))
  return _run(uidx, iidx, userEmbd, ubias2, ibias2, ob16)


# speedup vs baseline: 1.5904x; 1.0039x over previous
import jax
import jax.numpy as jnp
from jax import lax
from jax.experimental import pallas as pl
from jax.experimental.pallas import tpu as pltpu
from jax.experimental.pallas import tpu_sc as plsc

BATCH = 16384
EMBED = 64
LANES = 16
N_PER = BATCH // 32
WAVE = 16
N_WAVES = N_PER // WAVE
BCHUNK = 128


def _sc_body(uidx_hbm, iidx_hbm, emb_hbm, ubias_hbm, ibias_hbm, ob_hbm,
             out_hbm,
             uidx_v, iidx_v, u_blk, v_blk,
             ub_v, ib_v, ob_v, out_v, bias_sem, usems, vsems):
  nc = 2
  wid = lax.axis_index("s") * nc + lax.axis_index("c")
  base = wid * N_PER

  pltpu.sync_copy(uidx_hbm.at[pl.ds(base, N_PER)], uidx_v)
  pltpu.sync_copy(iidx_hbm.at[pl.ds(base, N_PER)], iidx_v)
  pltpu.sync_copy(ob_hbm, ob_v)

  for c in range(N_PER // BCHUNK):
    sl = pl.ds(c * BCHUNK, BCHUNK)
    pltpu.make_async_copy(ubias_hbm.at[0].at[uidx_v.at[sl]], ub_v.at[sl],
                          bias_sem).start()
    pltpu.make_async_copy(ibias_hbm.at[0].at[iidx_v.at[sl]], ib_v.at[sl],
                          bias_sem).start()

  def fire(w, buf):
    sl = pl.ds(w * WAVE, WAVE)
    uq = lax.shift_right_logical(uidx_v[sl], 3) * 8
    iq = lax.shift_right_logical(iidx_v[sl], 3) * 8
    for j in range(WAVE):
      rqu = pl.multiple_of(uq[j], 8)
      rqi = pl.multiple_of(iq[j], 8)
      pltpu.make_async_copy(emb_hbm.at[pl.ds(rqu, 8)], u_blk.at[buf, j],
                            usems.at[buf]).start()
      pltpu.make_async_copy(emb_hbm.at[pl.ds(rqi, 8)], v_blk.at[buf, j],
                            vsems.at[buf]).start()

  def wait(buf):
    def lp(j, _):
      pltpu.make_async_copy(emb_hbm.at[pl.ds(0, 8)], u_blk.at[buf, j],
                            usems.at[buf]).wait()
      pltpu.make_async_copy(emb_hbm.at[pl.ds(0, 8)], v_blk.at[buf, j],
                            vsems.at[buf]).wait()
      return 0

    lax.fori_loop(0, WAVE, lp, 0)

  lane = lax.iota(jnp.int32, LANES)

  def compute(w, buf):
    def group_body(g, _):
      sl = pl.ds(w * WAVE + g * LANES, LANES)
      blk = g * LANES + lane
      usub = uidx_v[sl] & 7
      isub = iidx_v[sl] & 7
      acc = jnp.zeros((LANES,), jnp.float32)

      def f_body(f, acc):
        cols = jnp.zeros((LANES,), jnp.int32) + f
        u = plsc.load_gather(u_blk.at[buf], [blk, usub, cols])
        v = plsc.load_gather(v_blk.at[buf], [blk, isub, cols])
        return acc + u * v

      acc = lax.fori_loop(0, EMBED, f_body, acc, unroll=8)
      out_v[sl] = acc
      return 0

    lax.fori_loop(0, WAVE // LANES, group_body, 0)

  fire(0, 0)

  def wave_body(w, _):
    @pl.when(w + 1 < N_WAVES)
    def _():
      fire(w + 1, (w + 1) % 2)
    wait(w % 2)
    compute(w, w % 2)
    return 0

  lax.fori_loop(0, N_WAVES, wave_body, 0)

  for c in range(N_PER // BCHUNK):
    sl = pl.ds(c * BCHUNK, BCHUNK)
    pltpu.make_async_copy(ubias_hbm.at[0].at[uidx_v.at[sl]], ub_v.at[sl],
                          bias_sem).wait()
    pltpu.make_async_copy(ibias_hbm.at[0].at[iidx_v.at[sl]], ib_v.at[sl],
                          bias_sem).wait()
  ob = ob_v[...]

  def bias_body(j, _):
    sl = pl.ds(j * LANES, LANES)
    out_v[sl] = out_v[sl] + ub_v[sl] + ib_v[sl] + ob
    return 0

  lax.fori_loop(0, N_PER // LANES, bias_body, 0)

  pltpu.sync_copy(out_v, out_hbm.at[pl.ds(base, N_PER)])


@jax.jit
def _run(uidx, iidx, emb, ubias2, ibias2, ob16):
  mesh = plsc.VectorSubcoreMesh(core_axis_name="c", subcore_axis_name="s")
  f = pl.kernel(
      _sc_body,
      out_type=jax.ShapeDtypeStruct((BATCH,), jnp.float32),
      mesh=mesh,
      scratch_types=[
          pltpu.VMEM((N_PER,), jnp.int32),
          pltpu.VMEM((N_PER,), jnp.int32),
          pltpu.VMEM((2, WAVE, 8, EMBED), jnp.float32),
          pltpu.VMEM((2, WAVE, 8, EMBED), jnp.float32),
          pltpu.VMEM((N_PER,), jnp.float32),
          pltpu.VMEM((N_PER,), jnp.float32),
          pltpu.VMEM((LANES,), jnp.float32),
          pltpu.VMEM((N_PER,), jnp.float32),
          pltpu.SemaphoreType.DMA,
          pltpu.SemaphoreType.DMA((2,)),
          pltpu.SemaphoreType.DMA((2,)),
      ],
      compiler_params=pltpu.CompilerParams(
          needs_layout_passes=False, use_tc_tiling_on_sc=True),
  )
  return f(uidx, iidx, emb, ubias2, ibias2, ob16)


def kernel(user_idx, item_idx, userEmbd, userBias, itemBias, overall_bias):
  uidx = user_idx.astype(jnp.int32)
  iidx = item_idx.astype(jnp.int32)
  ubias2 = userBias.T
  ibias2 = itemBias.T
  ob16 = jnp.broadcast_to(overall_bias.astype(jnp.float32), (LANES,))
  return _run(uidx, iidx, userEmbd, ubias2, ibias2, ob16)


# speedup vs baseline: 1.6184x; 1.0176x over previous
import jax
import jax.numpy as jnp
from jax import lax
from jax.experimental import pallas as pl
from jax.experimental.pallas import tpu as pltpu
from jax.experimental.pallas import tpu_sc as plsc

BATCH = 16384
EMBED = 64
LANES = 16
N_PER = BATCH // 32
WAVE = 16
N_WAVES = N_PER // WAVE
BCHUNK = 128


def _sc_body(uidx_hbm, iidx_hbm, emb_hbm, ubias_hbm, ibias_hbm, ob_hbm,
             out_hbm,
             uidx_v, iidx_v, u_blk, v_blk,
             ub_v, ib_v, ob_v, out_v, bias_sem, usems, vsems):
  nc = 2
  wid = lax.axis_index("s") * nc + lax.axis_index("c")
  base = wid * N_PER

  pltpu.sync_copy(uidx_hbm.at[pl.ds(base, N_PER)], uidx_v)
  pltpu.sync_copy(iidx_hbm.at[pl.ds(base, N_PER)], iidx_v)
  pltpu.sync_copy(ob_hbm, ob_v)

  for c in range(N_PER // BCHUNK):
    sl = pl.ds(c * BCHUNK, BCHUNK)
    pltpu.make_async_copy(ubias_hbm.at[0].at[uidx_v.at[sl]], ub_v.at[sl],
                          bias_sem).start()
    pltpu.make_async_copy(ibias_hbm.at[0].at[iidx_v.at[sl]], ib_v.at[sl],
                          bias_sem).start()

  def fire(w, buf):
    sl = pl.ds(w * WAVE, WAVE)
    uq = lax.shift_right_logical(uidx_v[sl], 3) * 8
    iq = lax.shift_right_logical(iidx_v[sl], 3) * 8
    for j in range(WAVE):
      rqu = pl.multiple_of(uq[j], 8)
      rqi = pl.multiple_of(iq[j], 8)
      pltpu.make_async_copy(emb_hbm.at[pl.ds(rqu, 8)], u_blk.at[buf, j],
                            usems.at[buf]).start()
      pltpu.make_async_copy(emb_hbm.at[pl.ds(rqi, 8)], v_blk.at[buf, j],
                            vsems.at[buf]).start()

  def wait(buf):
    def lp(j, _):
      pltpu.make_async_copy(emb_hbm.at[pl.ds(0, 8)], u_blk.at[buf, j],
                            usems.at[buf]).wait()
      pltpu.make_async_copy(emb_hbm.at[pl.ds(0, 8)], v_blk.at[buf, j],
                            vsems.at[buf]).wait()
      return 0

    lax.fori_loop(0, WAVE, lp, 0)

  lane = lax.iota(jnp.int32, LANES)

  def compute(w, buf):
    def group_body(g, _):
      sl = pl.ds(w * WAVE + g * LANES, LANES)
      blk = g * LANES + lane
      usub = uidx_v[sl] & 7
      isub = iidx_v[sl] & 7
      acc = jnp.zeros((LANES,), jnp.float32)

      def f_body(f, acc):
        cols = jnp.zeros((LANES,), jnp.int32) + f
        u = plsc.load_gather(u_blk.at[buf], [blk, usub, cols])
        v = plsc.load_gather(v_blk.at[buf], [blk, isub, cols])
        return acc + u * v

      acc = lax.fori_loop(0, EMBED, f_body, acc, unroll=8)
      out_v[sl] = acc
      return 0

    lax.fori_loop(0, WAVE // LANES, group_body, 0)

  fire(0, 0)
  fire(1, 1)

  def wave_body(w, _):
    @pl.when(w + 2 < N_WAVES)
    def _():
      fire(w + 2, (w + 2) % 3)
    wait(w % 3)
    compute(w, w % 3)
    return 0

  lax.fori_loop(0, N_WAVES, wave_body, 0)

  for c in range(N_PER // BCHUNK):
    sl = pl.ds(c * BCHUNK, BCHUNK)
    pltpu.make_async_copy(ubias_hbm.at[0].at[uidx_v.at[sl]], ub_v.at[sl],
                          bias_sem).wait()
    pltpu.make_async_copy(ibias_hbm.at[0].at[iidx_v.at[sl]], ib_v.at[sl],
                          bias_sem).wait()
  ob = ob_v[...]

  def bias_body(j, _):
    sl = pl.ds(j * LANES, LANES)
    out_v[sl] = out_v[sl] + ub_v[sl] + ib_v[sl] + ob
    return 0

  lax.fori_loop(0, N_PER // LANES, bias_body, 0)

  pltpu.sync_copy(out_v, out_hbm.at[pl.ds(base, N_PER)])


@jax.jit
def _run(uidx, iidx, emb, ubias2, ibias2, ob16):
  mesh = plsc.VectorSubcoreMesh(core_axis_name="c", subcore_axis_name="s")
  f = pl.kernel(
      _sc_body,
      out_type=jax.ShapeDtypeStruct((BATCH,), jnp.float32),
      mesh=mesh,
      scratch_types=[
          pltpu.VMEM((N_PER,), jnp.int32),
          pltpu.VMEM((N_PER,), jnp.int32),
          pltpu.VMEM((3, WAVE, 8, EMBED), jnp.float32),
          pltpu.VMEM((3, WAVE, 8, EMBED), jnp.float32),
          pltpu.VMEM((N_PER,), jnp.float32),
          pltpu.VMEM((N_PER,), jnp.float32),
          pltpu.VMEM((LANES,), jnp.float32),
          pltpu.VMEM((N_PER,), jnp.float32),
          pltpu.SemaphoreType.DMA,
          pltpu.SemaphoreType.DMA((3,)),
          pltpu.SemaphoreType.DMA((3,)),
      ],
      compiler_params=pltpu.CompilerParams(
          needs_layout_passes=False, use_tc_tiling_on_sc=True),
  )
  return f(uidx, iidx, emb, ubias2, ibias2, ob16)


def kernel(user_idx, item_idx, userEmbd, userBias, itemBias, overall_bias):
  uidx = user_idx.astype(jnp.int32)
  iidx = item_idx.astype(jnp.int32)
  ubias2 = userBias.T
  ibias2 = itemBias.T
  ob16 = jnp.broadcast_to(overall_bias.astype(jnp.float32), (LANES,))
  return _run(uidx, iidx, userEmbd, ubias2, ibias2, ob16)
